# baseline (speedup 1.0000x reference)
import functools

import jax
import jax.numpy as jnp
import numpy as np
from jax import lax
from jax.experimental import pallas as pl
from jax.experimental.pallas import tpu as pltpu
from jax.experimental.pallas import tpu_sc as plsc

_CLIP = np.array([(1, 0, 1), (2, 0, 1), (3, 0, -1), (4, 0, -1), (5, 0, -1),
                  (6, 0, -1), (7, 0, -1), (8, 0, -1), (9, 0, -1), (12, 0, -1),
                  (13, 1, 1), (14, 1, -1), (16, 1, 1), (17, 1, -1),
                  (18, 1, 1), (19, 1, -1)], dtype=np.int64)
_ZERO = np.array([(10, 0), (10, 1), (10, 2), (11, 0), (11, 1), (11, 2),
                  (15, 0), (15, 1), (15, 2), (20, 1), (21, 1)], dtype=np.int64)

_N_ROWS = 16384
_N_COLS = 63
_N_TERMS = len(_CLIP) + len(_ZERO)
_SCALE = 1.0 / (_N_ROWS * _N_TERMS)

_P_COLS = tuple(int((j - 1) * 3 + a) for j, a, s in _CLIP if s > 0)
_N_COLS_NEG = tuple(int((j - 1) * 3 + a) for j, a, s in _CLIP if s < 0)
_Z_COLS = tuple(int((j - 1) * 3 + a) for j, a in _ZERO)
_USED = ([(c, 0) for c in _P_COLS] + [(c, 1) for c in _N_COLS_NEG]
         + [(c, 2) for c in _Z_COLS])

_SC_ROWS = _N_ROWS // 2
_TC_ROWS = _N_ROWS - _SC_ROWS

_NW = 32
_COLS_PER_W = _SC_ROWS // _NW
_VECS = _COLS_PER_W // 16

_mesh = plsc.VectorSubcoreMesh(core_axis_name="c", subcore_axis_name="s")


def _tc_weights():
    sa = np.zeros((_N_COLS, 1), np.float64)
    sb = np.zeros((_N_COLS, 1), np.float64)
    for c in _P_COLS:
        sa[c, 0] = 1.0
    for c in _N_COLS_NEG:
        sa[c, 0] = -1.0
    for c in _Z_COLS:
        sa[c, 0] = 1.0
        sb[c, 0] = -1.0
    return ((sa * _SCALE).astype(np.float32), (sb * _SCALE).astype(np.float32))


_SA_TC, _SB_TC = _tc_weights()
_TC_BLK = 2048
_TC_GRID = _TC_ROWS // _TC_BLK


@functools.partial(
    pl.kernel,
    out_type=jax.ShapeDtypeStruct((_NW * 16,), jnp.float32),
    mesh=_mesh,
    scratch_types=[
        pltpu.VMEM((_N_TERMS * _COLS_PER_W,), jnp.float32),
        pltpu.VMEM((16,), jnp.float32),
        pltpu.SemaphoreType.DMA,
    ],
)
def _sc_partial_sums(pose_t_hbm, out_hbm, x_v, acc_v, sem):
    wid = lax.axis_index("s") * 2 + lax.axis_index("c")
    col0 = wid * _COLS_PER_W

    handles = []
    for k, (c, _) in enumerate(_USED):
        handles.append(pltpu.async_copy(
            pose_t_hbm.at[c, pl.ds(col0, _COLS_PER_W)],
            x_v.at[pl.ds(k * _COLS_PER_W, _COLS_PER_W)],
            sem,
        ))

    zero = jnp.zeros((16,), jnp.float32)

    def make_body(rows):
        def body(j, accs):
            p0, p1, n0, n1, z0, z1 = accs
            off = j * 16
            res = [[], [], []]
            for k, kind in rows:
                x = x_v[pl.ds(k * _COLS_PER_W + off, 16)]
                if kind == 0:
                    res[0].append(jnp.maximum(x, 0.0))
                elif kind == 1:
                    res[1].append(jnp.minimum(x, 0.0))
                else:
                    res[2].append(jnp.abs(x))
            p0 = p0 + sum(res[0][0::2], zero)
            p1 = p1 + sum(res[0][1::2], zero)
            n0 = n0 + sum(res[1][0::2], zero)
            n1 = n1 + sum(res[1][1::2], zero)
            z0 = z0 + sum(res[2][0::2], zero)
            z1 = z1 + sum(res[2][1::2], zero)
            return (p0, p1, n0, n1, z0, z1)
        return body

    accs = (zero,) * 6
    todo = [(k, kind) for k, (_, kind) in enumerate(_USED)]
    for g in range(0, _N_TERMS, 9):
        for h in handles[g:g + 9]:
            h.wait()
        accs = lax.fori_loop(0, _VECS, make_body(todo[g:g + 9]), accs)
    p0, p1, n0, n1, z0, z1 = accs
    acc = ((p0 + p1) - (n0 + n1) + (z0 + z1)) * jnp.float32(_SCALE)
    acc_v[...] = acc
    pltpu.sync_copy(acc_v, out_hbm.at[pl.ds(wid * 16, 16)])


def _tc_body(x_ref, sa_ref, sb_ref, out_ref):
    @pl.when(pl.program_id(0) == 0)
    def _():
        out_ref[...] = jnp.zeros((1, 1), jnp.float32)

    x = x_ref[...]
    out_ref[...] += jnp.sum(jnp.maximum(x * sa_ref[...], x * sb_ref[...]),
                            keepdims=True)


_tc_partial = pl.pallas_call(
    _tc_body,
    out_shape=jax.ShapeDtypeStruct((1, 1), jnp.float32),
    grid=(_TC_GRID,),
    in_specs=[
        pl.BlockSpec((_N_COLS, _TC_BLK),
                     lambda j: (0, _SC_ROWS // _TC_BLK + j)),
        pl.BlockSpec((_N_COLS, 1), lambda j: (0, 0)),
        pl.BlockSpec((_N_COLS, 1), lambda j: (0, 0)),
    ],
    out_specs=pl.BlockSpec((1, 1), lambda j: (0, 0)),
)


def kernel(pose):
    pose_t = pose.T
    sc_parts = _sc_partial_sums(pose_t)
    tc_part = _tc_partial(pose_t, jnp.asarray(_SA_TC), jnp.asarray(_SB_TC))
    return jnp.sum(sc_parts) + tc_part[0, 0]


# speedup vs baseline: 1.0356x; 1.0356x over previous
import functools

import jax
import jax.numpy as jnp
import numpy as np
from jax import lax
from jax.experimental import pallas as pl
from jax.experimental.pallas import tpu as pltpu
from jax.experimental.pallas import tpu_sc as plsc

_CLIP = np.array([(1, 0, 1), (2, 0, 1), (3, 0, -1), (4, 0, -1), (5, 0, -1),
                  (6, 0, -1), (7, 0, -1), (8, 0, -1), (9, 0, -1), (12, 0, -1),
                  (13, 1, 1), (14, 1, -1), (16, 1, 1), (17, 1, -1),
                  (18, 1, 1), (19, 1, -1)], dtype=np.int64)
_ZERO = np.array([(10, 0), (10, 1), (10, 2), (11, 0), (11, 1), (11, 2),
                  (15, 0), (15, 1), (15, 2), (20, 1), (21, 1)], dtype=np.int64)

_N_ROWS = 16384
_N_COLS = 63
_N_TERMS = len(_CLIP) + len(_ZERO)
_SCALE = 1.0 / (_N_ROWS * _N_TERMS)

_P_COLS = tuple(int((j - 1) * 3 + a) for j, a, s in _CLIP if s > 0)
_N_COLS_NEG = tuple(int((j - 1) * 3 + a) for j, a, s in _CLIP if s < 0)
_Z_COLS = tuple(int((j - 1) * 3 + a) for j, a in _ZERO)
_USED = ([(c, 0) for c in _P_COLS] + [(c, 1) for c in _N_COLS_NEG]
         + [(c, 2) for c in _Z_COLS])

_SC_ROWS = _N_ROWS

_NW = 32
_COLS_PER_W = _SC_ROWS // _NW
_VECS = _COLS_PER_W // 16

_mesh = plsc.VectorSubcoreMesh(core_axis_name="c", subcore_axis_name="s")


@functools.partial(
    pl.kernel,
    out_type=jax.ShapeDtypeStruct((_NW * 16,), jnp.float32),
    mesh=_mesh,
    scratch_types=[
        pltpu.VMEM((_N_TERMS * _COLS_PER_W,), jnp.float32),
        pltpu.VMEM((16,), jnp.float32),
        pltpu.SemaphoreType.DMA,
    ],
)
def _sc_partial_sums(pose_t_hbm, out_hbm, x_v, acc_v, sem):
    wid = lax.axis_index("s") * 2 + lax.axis_index("c")
    col0 = wid * _COLS_PER_W

    handles = []
    for k, (c, _) in enumerate(_USED):
        handles.append(pltpu.async_copy(
            pose_t_hbm.at[c, pl.ds(col0, _COLS_PER_W)],
            x_v.at[pl.ds(k * _COLS_PER_W, _COLS_PER_W)],
            sem,
        ))

    zero = jnp.zeros((16,), jnp.float32)

    def make_body(rows):
        def body(j, accs):
            p0, p1, n0, n1, z0, z1 = accs
            off = j * 16
            res = [[], [], []]
            for k, kind in rows:
                x = x_v[pl.ds(k * _COLS_PER_W + off, 16)]
                if kind == 0:
                    res[0].append(jnp.maximum(x, 0.0))
                elif kind == 1:
                    res[1].append(jnp.minimum(x, 0.0))
                else:
                    res[2].append(jnp.abs(x))
            p0 = p0 + sum(res[0][0::2], zero)
            p1 = p1 + sum(res[0][1::2], zero)
            n0 = n0 + sum(res[1][0::2], zero)
            n1 = n1 + sum(res[1][1::2], zero)
            z0 = z0 + sum(res[2][0::2], zero)
            z1 = z1 + sum(res[2][1::2], zero)
            return (p0, p1, n0, n1, z0, z1)
        return body

    accs = (zero,) * 6
    todo = [(k, kind) for k, (_, kind) in enumerate(_USED)]
    for g in range(0, _N_TERMS, 9):
        for h in handles[g:g + 9]:
            h.wait()
        accs = lax.fori_loop(0, _VECS, make_body(todo[g:g + 9]), accs)
    p0, p1, n0, n1, z0, z1 = accs
    acc = ((p0 + p1) - (n0 + n1) + (z0 + z1)) * jnp.float32(_SCALE)
    acc_v[...] = acc
    pltpu.sync_copy(acc_v, out_hbm.at[pl.ds(wid * 16, 16)])


def kernel(pose):
    return jnp.sum(_sc_partial_sums(pose.T))
